# baseline (speedup 1.0000x reference)
import functools

import jax
import jax.numpy as jnp
from jax import lax
from jax.experimental import pallas as pl
from jax.experimental.pallas import tpu as pltpu
from jax.experimental.pallas import tpu_sc as plsc

EMBED = 64
GB = 128
NUM_CORES = 2
NUM_SUBCORES = 16
NW = NUM_CORES * NUM_SUBCORES
NG = 4
NBUF = 2


def _sc_gather_t(x_lmaj, table, bsz, seq):
    n = x_lmaj.shape[0]
    per_w = n // NW
    chunk = NG * GB
    steps = per_w // chunk
    groups_per_l = bsz // GB
    assert n == bsz * seq and n % NW == 0 and per_w % chunk == 0
    assert steps % 1 == 0 and bsz % GB == 0

    mesh = plsc.VectorSubcoreMesh(core_axis_name="c", subcore_axis_name="s")

    @functools.partial(
        pl.kernel,
        mesh=mesh,
        out_type=jax.ShapeDtypeStruct((seq, EMBED, bsz), jnp.float32),
        scratch_types=[
            pltpu.VMEM((per_w,), jnp.int32),
            pltpu.VMEM((NBUF, chunk, EMBED), jnp.float32),
            pltpu.VMEM((NG, EMBED, GB), jnp.float32),
        ]
        + [pltpu.SemaphoreType.DMA] * (NBUF + NG),
        compiler_params=pltpu.CompilerParams(
            use_tc_tiling_on_sc=False, needs_layout_passes=False),
    )
    def k(idx_hbm, table_hbm, out_hbm, idx_all, rows, tbuf, *sems):
        gsems = sems[:NBUF]
        tsems = sems[NBUF:]
        wid = lax.axis_index("s") * NUM_CORES + lax.axis_index("c")
        base = wid * per_w
        base_g = wid * (per_w // GB)
        pltpu.sync_copy(idx_hbm.at[pl.ds(base, per_w)], idx_all)

        def g_start(c, b):
            pltpu.async_copy(
                table_hbm.at[idx_all.at[pl.ds(c * chunk, chunk)]],
                rows.at[b], gsems[b])

        def g_wait(b):
            pltpu.make_async_copy(
                table_hbm.at[idx_all.at[pl.ds(0, chunk)]],
                rows.at[b], gsems[b]).wait()

        def t_start(gidx, p):
            lpos = gidx // groups_per_l
            bt = gidx % groups_per_l
            pltpu.async_copy(
                tbuf.at[p],
                out_hbm.at[lpos, :, pl.ds(bt * GB, GB)], tsems[p])

        def t_wait(p):
            pltpu.make_async_copy(
                tbuf.at[p],
                out_hbm.at[0, :, pl.ds(0, GB)], tsems[p]).wait()

        lanes = lax.iota(jnp.int32, 16)

        def transpose_group(b, gi, p):
            rv = rows.at[b]
            tv = tbuf.at[p]

            def jbody(j, carry):
                jv = jnp.zeros((16,), jnp.int32) + j
                for m in range(GB // 16):
                    riv = lanes + (gi * GB + m * 16)
                    val = plsc.load_gather(rv, [riv, jv])
                    tv[j, pl.ds(m * 16, 16)] = val
                return carry

            lax.fori_loop(0, EMBED, jbody, 0)

        for b in range(NBUF):
            g_start(b, b)

        def body(c, carry):
            b = lax.rem(c, NBUF)

            def work(bstat):
                g_wait(bstat)
                for gi in range(NG):

                    @pl.when(c >= 1)
                    def _():
                        t_wait(gi)

                    transpose_group(bstat, gi, gi)
                    t_start(base_g + c * NG + gi, gi)
                g_start(c + NBUF, bstat)

            @pl.when(b == 0)
            def _():
                work(0)

            @pl.when(b != 0)
            def _():
                work(1)

            return carry

        lax.fori_loop(0, steps - NBUF, body, 0)

        for cc in range(steps - NBUF, steps):
            b = cc % NBUF
            g_wait(b)
            for gi in range(NG):
                t_wait(gi)
                transpose_group(b, gi, gi)
                t_start(base_g + cc * NG + gi, gi)
        for gi in range(NG):
            t_wait(gi)

    return k(x_lmaj, table)


def kernel(x, table):
    b, l = x.shape
    x_lmaj = x.T.reshape(b * l).astype(jnp.int32)
    out = _sc_gather_t(x_lmaj, table, b, l)
    return jnp.transpose(out, (2, 0, 1))


# speedup vs baseline: 1.6243x; 1.6243x over previous
import functools

import jax
import jax.numpy as jnp
from jax import lax
from jax.experimental import pallas as pl
from jax.experimental.pallas import tpu as pltpu
from jax.experimental.pallas import tpu_sc as plsc

EMBED = 64
NUM_CORES = 2
NUM_SUBCORES = 16
NW = NUM_CORES * NUM_SUBCORES
CHUNK = 512
NBUF = 2


def _sc_gather(x_flat, table):
    n = x_flat.shape[0]
    per_w = n // NW
    steps = per_w // CHUNK
    assert n % NW == 0 and per_w % CHUNK == 0 and steps % NBUF == 0

    mesh = plsc.VectorSubcoreMesh(core_axis_name="c", subcore_axis_name="s")

    @functools.partial(
        pl.kernel,
        mesh=mesh,
        out_type=jax.ShapeDtypeStruct((n, EMBED), jnp.float32),
        scratch_types=[
            pltpu.VMEM((per_w,), jnp.int32),
            pltpu.VMEM((NBUF, CHUNK, EMBED), jnp.float32),
        ]
        + [pltpu.SemaphoreType.DMA] * (2 * NBUF),
        compiler_params=pltpu.CompilerParams(use_tc_tiling_on_sc=False),
    )
    def k(idx_hbm, table_hbm, out_hbm, idx_all, rows, *sems):
        gsems = sems[:NBUF]
        ssems = sems[NBUF:]
        wid = lax.axis_index("s") * NUM_CORES + lax.axis_index("c")
        base = wid * per_w
        pltpu.sync_copy(idx_hbm.at[pl.ds(base, per_w)], idx_all)

        def g_start(c, b):
            pltpu.async_copy(
                table_hbm.at[idx_all.at[pl.ds(c * CHUNK, CHUNK)]],
                rows.at[b], gsems[b])

        def g_wait(b):
            pltpu.make_async_copy(
                table_hbm.at[idx_all.at[pl.ds(0, CHUNK)]],
                rows.at[b], gsems[b]).wait()

        def s_start(c, b):
            pltpu.async_copy(
                rows.at[b],
                out_hbm.at[pl.ds(base + c * CHUNK, CHUNK)], ssems[b])

        def s_wait(b):
            pltpu.make_async_copy(
                rows.at[b],
                out_hbm.at[pl.ds(base, CHUNK)], ssems[b]).wait()

        for b in range(NBUF):
            g_start(b, b)

        def body(g, carry):
            for b in range(NBUF):
                c = g * NBUF + b
                g_wait(b)
                s_start(c, b)
                s_wait(b)
                g_start(c + NBUF, b)
            return carry

        lax.fori_loop(0, steps // NBUF - 1, body, 0)

        c_last = steps - NBUF
        for b in range(NBUF):
            g_wait(b)
            s_start(c_last + b, b)
        for b in range(NBUF):
            s_wait(b)

    return k(x_flat, table)


def kernel(x, table):
    b, l = x.shape
    x_flat = x.reshape(b * l).astype(jnp.int32)
    out = _sc_gather(x_flat, table)
    return out.reshape(b, l, EMBED)


# speedup vs baseline: 1.6259x; 1.0010x over previous
import functools

import jax
import jax.numpy as jnp
from jax import lax
from jax.experimental import pallas as pl
from jax.experimental.pallas import tpu as pltpu
from jax.experimental.pallas import tpu_sc as plsc

EMBED = 64
NUM_CORES = 2
NUM_SUBCORES = 16
NW = NUM_CORES * NUM_SUBCORES
CHUNK = 320
NBUF = 5


def _sc_gather(x_flat, table):
    n = x_flat.shape[0]
    per_w = n // NW
    steps = per_w // CHUNK
    assert n % NW == 0 and per_w % CHUNK == 0 and steps % NBUF == 0

    mesh = plsc.VectorSubcoreMesh(core_axis_name="c", subcore_axis_name="s")

    @functools.partial(
        pl.kernel,
        mesh=mesh,
        out_type=jax.ShapeDtypeStruct((n, EMBED), jnp.float32),
        scratch_types=[
            pltpu.VMEM((per_w,), jnp.int32),
            pltpu.VMEM((NBUF, CHUNK, EMBED), jnp.float32),
        ]
        + [pltpu.SemaphoreType.DMA] * (2 * NBUF),
        compiler_params=pltpu.CompilerParams(use_tc_tiling_on_sc=False),
    )
    def k(idx_hbm, table_hbm, out_hbm, idx_all, rows, *sems):
        gsems = sems[:NBUF]
        ssems = sems[NBUF:]
        wid = lax.axis_index("s") * NUM_CORES + lax.axis_index("c")
        base = wid * per_w
        pltpu.sync_copy(idx_hbm.at[pl.ds(base, per_w)], idx_all)

        def g_start(c, b):
            pltpu.async_copy(
                table_hbm.at[idx_all.at[pl.ds(c * CHUNK, CHUNK)]],
                rows.at[b], gsems[b])

        def g_wait(b):
            pltpu.make_async_copy(
                table_hbm.at[idx_all.at[pl.ds(0, CHUNK)]],
                rows.at[b], gsems[b]).wait()

        def s_start(c, b):
            pltpu.async_copy(
                rows.at[b],
                out_hbm.at[pl.ds(base + c * CHUNK, CHUNK)], ssems[b])

        def s_wait(b):
            pltpu.make_async_copy(
                rows.at[b],
                out_hbm.at[pl.ds(base, CHUNK)], ssems[b]).wait()

        for b in range(NBUF):
            g_start(b, b)

        def body(g, carry):
            for b in range(NBUF):
                c = g * NBUF + b
                g_wait(b)
                s_start(c, b)
                s_wait(b)
                g_start(c + NBUF, b)
            return carry

        lax.fori_loop(0, steps // NBUF - 1, body, 0)

        c_last = steps - NBUF
        for b in range(NBUF):
            g_wait(b)
            s_start(c_last + b, b)
        for b in range(NBUF):
            s_wait(b)

    return k(x_flat, table)


def kernel(x, table):
    b, l = x.shape
    x_flat = x.reshape(b * l).astype(jnp.int32)
    out = _sc_gather(x_flat, table)
    return out.reshape(b, l, EMBED)
